# baseline (speedup 1.0000x reference)
import functools

import jax
import jax.numpy as jnp
from jax import lax
from jax.experimental import pallas as pl
from jax.experimental.pallas import tpu as pltpu
from jax.experimental.pallas import tpu_sc as plsc

N = 10000
E = 320000
D = 128

NC = 2
NS = 16
NW = NC * NS

CH = 128
E2 = 327680
EPW = E2 // NW
NCH = EPW // CH
C4 = 64
NCH4 = EPW // C4
BCH = 16
NBL = NCH4 // BCH
NPAD = 10240
SEG = NPAD // NS

_MESH = plsc.VectorSubcoreMesh(core_axis_name="c", subcore_axis_name="s",
                               num_cores=NC, num_subcores=NS)


@functools.partial(
    pl.kernel,
    out_type=jax.ShapeDtypeStruct((NC, NPAD), jnp.float32),
    mesh=_MESH,
    compiler_params=pltpu.CompilerParams(needs_layout_passes=False),
    scratch_types=[
        pltpu.VMEM((NCH, CH), jnp.int32),
        pltpu.VMEM((NCH, CH), jnp.float32),
        pltpu.VMEM((SEG,), jnp.float32),
        pltpu.VMEM_SHARED((NPAD,), jnp.float32),
        pltpu.SemaphoreType.DMA,
    ],
)
def _deg_kernel(dst_hbm, w_hbm, deg_hbm, idx_v, w_v, buf_v, acc_sh, dsem):
    c = lax.axis_index("c")
    s = lax.axis_index("s")
    wid = c * NS + s

    pltpu.sync_copy(dst_hbm.at[wid], idx_v)
    pltpu.sync_copy(w_hbm.at[wid], w_v)

    def zero(i, _):
        buf_v[pl.ds(i * 16, 16)] = jnp.zeros((16,), jnp.float32)
        return 0
    lax.fori_loop(0, SEG // 16, zero, 0)
    pltpu.sync_copy(buf_v, acc_sh.at[pl.ds(s * SEG, SEG)])
    plsc.subcore_barrier()

    def body(g, _):
        for i in range(16):
            j = g * 16 + i
            pltpu.async_copy(w_v.at[j], acc_sh.at[idx_v.at[j]], dsem,
                             add=True)
        for i in range(16):
            pltpu.make_async_copy(w_v.at[0], acc_sh.at[idx_v.at[0]],
                                  dsem).wait()
        return 0
    lax.fori_loop(0, NCH // 16, body, 0)
    plsc.subcore_barrier()

    pltpu.sync_copy(acc_sh.at[pl.ds(s * SEG, SEG)],
                    deg_hbm.at[c, pl.ds(s * SEG, SEG)])


@functools.partial(
    pl.kernel,
    out_type=jax.ShapeDtypeStruct((NC, NPAD, D), jnp.float32),
    mesh=_MESH,
    compiler_params=pltpu.CompilerParams(needs_layout_passes=False),
    scratch_types=[
        pltpu.VMEM((2, BCH, C4), jnp.int32),
        pltpu.VMEM((2, BCH, C4), jnp.int32),
        pltpu.VMEM((2, BCH, C4), jnp.float32),
        pltpu.VMEM((C4, D), jnp.float32),
        pltpu.VMEM((C4, D), jnp.float32),
        pltpu.VMEM((C4, D), jnp.float32),
        pltpu.VMEM((C4, D), jnp.float32),
        pltpu.VMEM_SHARED((NPAD, D), jnp.float32),
        pltpu.SemaphoreType.DMA,
        pltpu.SemaphoreType.DMA,
        pltpu.SemaphoreType.DMA,
        pltpu.SemaphoreType.DMA,
        pltpu.SemaphoreType.DMA,
        pltpu.SemaphoreType.DMA,
        pltpu.SemaphoreType.DMA,
        pltpu.SemaphoreType.DMA,
        pltpu.SemaphoreType.DMA,
    ],
)
def _agg_kernel(hp_hbm, src_hbm, dst_hbm, w_hbm, out_hbm,
                sblk_v, dblk_v, wblk_v, rows0, rows1, rows2, rows3, acc_sh,
                esem, g0sem, g1sem, g2sem, g3sem, s0sem, s1sem, s2sem, s3sem):
    c = lax.axis_index("c")
    s = lax.axis_index("s")
    wid = c * NS + s
    rows = (rows0, rows1, rows2, rows3)
    gsems = (g0sem, g1sem, g2sem, g3sem)
    ssems = (s0sem, s1sem, s2sem, s3sem)

    def scale_rows(rows_v, pb, k):
        def scale(g, _):
            ws16 = wblk_v[pb, k, pl.ds(g * 16, 16)]
            for r in range(16):
                row = g * 16 + r
                wbc = lax.gather(
                    ws16, jnp.full((16, 1), r, jnp.int32),
                    lax.GatherDimensionNumbers(
                        offset_dims=(), collapsed_slice_dims=(0,),
                        start_index_map=(0,)),
                    (1,), mode=lax.GatherScatterMode.PROMISE_IN_BOUNDS)
                for f in range(D // 16):
                    rows_v[row, pl.ds(f * 16, 16)] = (
                        rows_v[row, pl.ds(f * 16, 16)] * wbc)
            return 0
        lax.fori_loop(0, C4 // 16, scale, 0)

    def wait_edge_block():
        for _ in range(3):
            pltpu.make_async_copy(src_hbm.at[wid, 0], sblk_v.at[0], esem).wait()

    def wait_gather(i):
        pltpu.make_async_copy(hp_hbm.at[sblk_v.at[0, 0]], rows[i],
                              gsems[i]).wait()

    def wait_scatter(i):
        pltpu.make_async_copy(rows[i], acc_sh.at[dblk_v.at[0, 0]],
                              ssems[i]).wait()

    pltpu.async_copy(src_hbm.at[wid, 0], sblk_v.at[0], esem)
    pltpu.async_copy(dst_hbm.at[wid, 0], dblk_v.at[0], esem)
    pltpu.async_copy(w_hbm.at[wid, 0], wblk_v.at[0], esem)

    def zrow(r, _):
        for f in range(D // 16):
            z = jnp.zeros((16,), jnp.float32)
            rows0[r, pl.ds(f * 16, 16)] = z
            rows1[r, pl.ds(f * 16, 16)] = z
            rows2[r, pl.ds(f * 16, 16)] = z
            rows3[r, pl.ds(f * 16, 16)] = z
        return 0
    lax.fori_loop(0, C4, zrow, 0)
    for q in range(SEG // C4):
        pltpu.sync_copy(rows0, acc_sh.at[pl.ds(s * SEG + q * C4, C4)])
    wait_edge_block()
    plsc.subcore_barrier()

    pltpu.async_copy(hp_hbm.at[sblk_v.at[0, 0]], rows0, g0sem)
    pltpu.async_copy(hp_hbm.at[sblk_v.at[0, 1]], rows1, g1sem)
    pltpu.async_copy(rows2, acc_sh.at[dblk_v.at[0, 0]], s2sem, add=True)
    pltpu.async_copy(rows3, acc_sh.at[dblk_v.at[0, 0]], s3sem, add=True)

    def block(b, _):
        pb = lax.rem(b, 2)

        @pl.when(b < NBL - 1)
        def _():
            nb = 1 - pb
            pltpu.async_copy(src_hbm.at[wid, b + 1], sblk_v.at[nb], esem)
            pltpu.async_copy(dst_hbm.at[wid, b + 1], dblk_v.at[nb], esem)
            pltpu.async_copy(w_hbm.at[wid, b + 1], wblk_v.at[nb], esem)

        def quad(q, _):
            for i in range(4):
                k = 4 * q + i
                ni = (i + 2) % 4
                wait_scatter(ni)
                if i < 2:
                    pltpu.async_copy(hp_hbm.at[sblk_v.at[pb, k + 2]],
                                     rows[ni], gsems[ni])
                else:
                    @pl.when(q < BCH // 4 - 1)
                    def _():
                        pltpu.async_copy(hp_hbm.at[sblk_v.at[pb, k + 2]],
                                         rows[ni], gsems[ni])
                wait_gather(i)
                scale_rows(rows[i], pb, k)
                pltpu.async_copy(rows[i], acc_sh.at[dblk_v.at[pb, k]],
                                 ssems[i], add=True)
            return 0
        lax.fori_loop(0, BCH // 4, quad, 0)

        @pl.when(b < NBL - 1)
        def _():
            nb = 1 - pb
            wait_edge_block()
            pltpu.async_copy(hp_hbm.at[sblk_v.at[nb, 0]], rows0, g0sem)
            pltpu.async_copy(hp_hbm.at[sblk_v.at[nb, 1]], rows1, g1sem)
        return 0
    lax.fori_loop(0, NBL, block, 0)

    wait_scatter(2)
    wait_scatter(3)
    plsc.subcore_barrier()

    pltpu.sync_copy(acc_sh.at[pl.ds(s * SEG, SEG)],
                    out_hbm.at[c, pl.ds(s * SEG, SEG)])


def _matmul_body(x_ref, w_ref, h_ref):
    h_ref[...] = jnp.dot(x_ref[...], w_ref[...],
                         preferred_element_type=jnp.float32)


def _prep_body(h_ref, dp_ref, hp_ref, dis_ref):
    deg = dp_ref[0, :N] + dp_ref[1, :N] + 1.0
    dis = jnp.where(deg > 0, lax.rsqrt(jnp.maximum(deg, 1e-12)), 0.0)
    hp_ref[...] = dis * h_ref[...]
    dis_ref[...] = dis


def _final_body(p_ref, hp_ref, dis_ref, b_ref, o_ref):
    dis = dis_ref[...]
    z = dis * (p_ref[0, :N] + p_ref[1, :N] + hp_ref[...]) + b_ref[...]
    o_ref[...] = jnp.where(z >= 0, z, 0.01 * z)


def kernel(x, edge_index, edge_attr, W, b):
    src = edge_index[0]
    dst = edge_index[1]

    npad = E2 - E
    pad_idx = (jnp.arange(npad, dtype=jnp.int32) * 13) % N
    src_p = jnp.concatenate([src, pad_idx])
    dst_p = jnp.concatenate([dst, pad_idx])
    w_p = jnp.concatenate([edge_attr, jnp.zeros((npad,), jnp.float32)])

    dst1 = dst_p.reshape(NW, NCH, CH)
    w1 = w_p.reshape(NW, NCH, CH)
    src3 = src_p.reshape(NW, NBL, BCH, C4)
    dst3 = dst_p.reshape(NW, NBL, BCH, C4)
    w3 = w_p.reshape(NW, NBL, BCH, C4)

    h = pl.pallas_call(
        _matmul_body,
        out_shape=jax.ShapeDtypeStruct((N, D), jnp.float32),
    )(x, W)

    deg_p = _deg_kernel(dst1, w1)

    hp, dis = pl.pallas_call(
        _prep_body,
        out_shape=[jax.ShapeDtypeStruct((N, D), jnp.float32),
                   jax.ShapeDtypeStruct((N, 1), jnp.float32)],
    )(h, deg_p.reshape(NC, NPAD, 1))

    part = _agg_kernel(hp, src3, dst3, w3)

    out = pl.pallas_call(
        _final_body,
        out_shape=jax.ShapeDtypeStruct((N, D), jnp.float32),
    )(part, hp, dis, b.reshape(1, D))
    return out


# speedup vs baseline: 1.0234x; 1.0234x over previous
import functools

import jax
import jax.numpy as jnp
from jax import lax
from jax.experimental import pallas as pl
from jax.experimental.pallas import tpu as pltpu
from jax.experimental.pallas import tpu_sc as plsc

N = 10000
E = 320000
D = 128

NC = 2
NS = 16
NW = NC * NS

CH = 128
E2 = 327680
EPW = E2 // NW
NCH = EPW // CH
C4 = 64
NCH4 = EPW // C4
BCH = 16
NBL = NCH4 // BCH
NPAD = 10240
SEG = NPAD // NS

_MESH = plsc.VectorSubcoreMesh(core_axis_name="c", subcore_axis_name="s",
                               num_cores=NC, num_subcores=NS)


@functools.partial(
    pl.kernel,
    out_type=jax.ShapeDtypeStruct((NC, NPAD), jnp.float32),
    mesh=_MESH,
    compiler_params=pltpu.CompilerParams(needs_layout_passes=False),
    scratch_types=[
        pltpu.VMEM((NCH, CH), jnp.int32),
        pltpu.VMEM((NCH, CH), jnp.float32),
        pltpu.VMEM((SEG,), jnp.float32),
        pltpu.VMEM_SHARED((NPAD,), jnp.float32),
        pltpu.SemaphoreType.DMA,
    ],
)
def _deg_kernel(dst_hbm, w_hbm, deg_hbm, idx_v, w_v, buf_v, acc_sh, dsem):
    c = lax.axis_index("c")
    s = lax.axis_index("s")
    wid = c * NS + s

    pltpu.sync_copy(dst_hbm.at[wid], idx_v)
    pltpu.sync_copy(w_hbm.at[wid], w_v)

    def zero(i, _):
        buf_v[pl.ds(i * 16, 16)] = jnp.zeros((16,), jnp.float32)
        return 0
    lax.fori_loop(0, SEG // 16, zero, 0)
    pltpu.sync_copy(buf_v, acc_sh.at[pl.ds(s * SEG, SEG)])
    plsc.subcore_barrier()

    def body(g, _):
        for i in range(16):
            j = g * 16 + i
            pltpu.async_copy(w_v.at[j], acc_sh.at[idx_v.at[j]], dsem,
                             add=True)
        for i in range(16):
            pltpu.make_async_copy(w_v.at[0], acc_sh.at[idx_v.at[0]],
                                  dsem).wait()
        return 0
    lax.fori_loop(0, NCH // 16, body, 0)
    plsc.subcore_barrier()

    pltpu.sync_copy(acc_sh.at[pl.ds(s * SEG, SEG)],
                    deg_hbm.at[c, pl.ds(s * SEG, SEG)])


@functools.partial(
    pl.kernel,
    out_type=jax.ShapeDtypeStruct((NC, NPAD, D), jnp.float32),
    mesh=_MESH,
    compiler_params=pltpu.CompilerParams(needs_layout_passes=False),
    scratch_types=[
        pltpu.VMEM((2, BCH, C4), jnp.int32),
        pltpu.VMEM((2, BCH, C4), jnp.int32),
        pltpu.VMEM((2, BCH, C4), jnp.float32),
        pltpu.VMEM((C4, D), jnp.float32),
        pltpu.VMEM((C4, D), jnp.float32),
        pltpu.VMEM((C4, D), jnp.float32),
        pltpu.VMEM((C4, D), jnp.float32),
        pltpu.VMEM_SHARED((NPAD, D), jnp.float32),
        pltpu.SemaphoreType.DMA,
        pltpu.SemaphoreType.DMA,
        pltpu.SemaphoreType.DMA,
        pltpu.SemaphoreType.DMA,
        pltpu.SemaphoreType.DMA,
        pltpu.SemaphoreType.DMA,
        pltpu.SemaphoreType.DMA,
        pltpu.SemaphoreType.DMA,
        pltpu.SemaphoreType.DMA,
    ],
)
def _agg_kernel(hp_hbm, src_hbm, dst_hbm, w_hbm, out_hbm,
                sblk_v, dblk_v, wblk_v, rows0, rows1, rows2, rows3, acc_sh,
                esem, g0sem, g1sem, g2sem, g3sem, s0sem, s1sem, s2sem, s3sem):
    c = lax.axis_index("c")
    s = lax.axis_index("s")
    wid = c * NS + s
    rows = (rows0, rows1, rows2, rows3)
    gsems = (g0sem, g1sem, g2sem, g3sem)
    ssems = (s0sem, s1sem, s2sem, s3sem)

    def scale_rows(rows_v, pb, k):
        def scale(g, _):
            ws16 = wblk_v[pb, k, pl.ds(g * 16, 16)]
            for r in range(16):
                row = g * 16 + r
                wbc = lax.gather(
                    ws16, jnp.full((16, 1), r, jnp.int32),
                    lax.GatherDimensionNumbers(
                        offset_dims=(), collapsed_slice_dims=(0,),
                        start_index_map=(0,)),
                    (1,), mode=lax.GatherScatterMode.PROMISE_IN_BOUNDS)
                for f in range(D // 16):
                    rows_v[row, pl.ds(f * 16, 16)] = (
                        rows_v[row, pl.ds(f * 16, 16)] * wbc)
            return 0
        lax.fori_loop(0, C4 // 16, scale, 0)

    def wait_edge_block():
        for _ in range(3):
            pltpu.make_async_copy(src_hbm.at[wid, 0], sblk_v.at[0], esem).wait()

    def wait_gather(i):
        pltpu.make_async_copy(hp_hbm.at[sblk_v.at[0, 0]], rows[i],
                              gsems[i]).wait()

    def wait_scatter(i):
        pltpu.make_async_copy(rows[i], acc_sh.at[dblk_v.at[0, 0]],
                              ssems[i]).wait()

    pltpu.async_copy(src_hbm.at[wid, 0], sblk_v.at[0], esem)
    pltpu.async_copy(dst_hbm.at[wid, 0], dblk_v.at[0], esem)
    pltpu.async_copy(w_hbm.at[wid, 0], wblk_v.at[0], esem)

    def zrow(r, _):
        for f in range(D // 16):
            z = jnp.zeros((16,), jnp.float32)
            rows0[r, pl.ds(f * 16, 16)] = z
            rows2[r, pl.ds(f * 16, 16)] = z
            rows3[r, pl.ds(f * 16, 16)] = z
        return 0
    lax.fori_loop(0, C4, zrow, 0)
    for q in range(SEG // C4):
        pltpu.async_copy(rows0, acc_sh.at[pl.ds(s * SEG + q * C4, C4)], g0sem)
    for q in range(SEG // C4):
        pltpu.make_async_copy(rows0, acc_sh.at[pl.ds(s * SEG, C4)],
                              g0sem).wait()
    wait_edge_block()
    plsc.subcore_barrier()

    pltpu.async_copy(hp_hbm.at[sblk_v.at[0, 0]], rows0, g0sem)
    pltpu.async_copy(hp_hbm.at[sblk_v.at[0, 1]], rows1, g1sem)
    pltpu.async_copy(rows2, acc_sh.at[dblk_v.at[0, 0]], s2sem, add=True)
    pltpu.async_copy(rows3, acc_sh.at[dblk_v.at[0, 0]], s3sem, add=True)

    def block(b, _):
        pb = lax.rem(b, 2)

        @pl.when(b < NBL - 1)
        def _():
            nb = 1 - pb
            pltpu.async_copy(src_hbm.at[wid, b + 1], sblk_v.at[nb], esem)
            pltpu.async_copy(dst_hbm.at[wid, b + 1], dblk_v.at[nb], esem)
            pltpu.async_copy(w_hbm.at[wid, b + 1], wblk_v.at[nb], esem)

        def quad(q, _):
            for i in range(4):
                k = 4 * q + i
                ni = (i + 2) % 4
                wait_scatter(ni)
                if i < 2:
                    pltpu.async_copy(hp_hbm.at[sblk_v.at[pb, k + 2]],
                                     rows[ni], gsems[ni])
                else:
                    @pl.when(q < BCH // 4 - 1)
                    def _():
                        pltpu.async_copy(hp_hbm.at[sblk_v.at[pb, k + 2]],
                                         rows[ni], gsems[ni])
                wait_gather(i)
                scale_rows(rows[i], pb, k)
                pltpu.async_copy(rows[i], acc_sh.at[dblk_v.at[pb, k]],
                                 ssems[i], add=True)
            return 0
        lax.fori_loop(0, BCH // 4, quad, 0)

        @pl.when(b < NBL - 1)
        def _():
            nb = 1 - pb
            wait_edge_block()
            pltpu.async_copy(hp_hbm.at[sblk_v.at[nb, 0]], rows0, g0sem)
            pltpu.async_copy(hp_hbm.at[sblk_v.at[nb, 1]], rows1, g1sem)
        return 0
    lax.fori_loop(0, NBL, block, 0)

    wait_scatter(2)
    wait_scatter(3)
    plsc.subcore_barrier()

    pltpu.sync_copy(acc_sh.at[pl.ds(s * SEG, SEG)],
                    out_hbm.at[c, pl.ds(s * SEG, SEG)])


def _prep_body(x_ref, w_ref, dp_ref, hp_ref, dis_ref):
    deg = dp_ref[0, :N] + dp_ref[1, :N] + 1.0
    dis = jnp.where(deg > 0, lax.rsqrt(jnp.maximum(deg, 1e-12)), 0.0)
    hp_ref[...] = dis * jnp.dot(x_ref[...], w_ref[...],
                                preferred_element_type=jnp.float32)
    dis_ref[...] = dis


def _final_body(p_ref, hp_ref, dis_ref, b_ref, o_ref):
    dis = dis_ref[...]
    z = dis * (p_ref[0, :N] + p_ref[1, :N] + hp_ref[...]) + b_ref[...]
    o_ref[...] = jnp.where(z >= 0, z, 0.01 * z)


def kernel(x, edge_index, edge_attr, W, b):
    src = edge_index[0]
    dst = edge_index[1]

    npad = E2 - E
    pad_idx = (jnp.arange(npad, dtype=jnp.int32) * 13) % N
    src_p = jnp.concatenate([src, pad_idx])
    dst_p = jnp.concatenate([dst, pad_idx])
    w_p = jnp.concatenate([edge_attr, jnp.zeros((npad,), jnp.float32)])

    dst1 = dst_p.reshape(NW, NCH, CH)
    w1 = w_p.reshape(NW, NCH, CH)
    src3 = src_p.reshape(NW, NBL, BCH, C4)
    dst3 = dst_p.reshape(NW, NBL, BCH, C4)
    w3 = w_p.reshape(NW, NBL, BCH, C4)

    deg_p = _deg_kernel(dst1, w1)

    hp, dis = pl.pallas_call(
        _prep_body,
        out_shape=[jax.ShapeDtypeStruct((N, D), jnp.float32),
                   jax.ShapeDtypeStruct((N, 1), jnp.float32)],
    )(x, W, deg_p.reshape(NC, NPAD, 1))

    part = _agg_kernel(hp, src3, dst3, w3)

    out = pl.pallas_call(
        _final_body,
        out_shape=jax.ShapeDtypeStruct((N, D), jnp.float32),
    )(part, hp, dis, b.reshape(1, D))
    return out
